# baseline (speedup 1.0000x reference)
import dataclasses
import functools

import jax
import jax.numpy as jnp
from jax import lax
from jax.experimental import pallas as pl
from jax.experimental.pallas import tpu as pltpu
from jax.experimental.pallas import tpu_sc as plsc

_N = 10000
_D = 128
_E = 320000
_NC = 2
_NS = 16
_NW = _NC * _NS
_CH = 96
_NCH = 105
_EP = _NW * _NCH * _CH
_NP = 10112
_RPT = _NP // _NS

_mesh = plsc.VectorSubcoreMesh(core_axis_name="c", subcore_axis_name="s")
_NV = 625
_NH = 10240

_cp = pltpu.CompilerParams()
if "needs_layout_passes" in pltpu.CompilerParams.__dataclass_fields__:
    _cp = dataclasses.replace(_cp, needs_layout_passes=False)


@functools.partial(
    pl.kernel,
    out_type=jax.ShapeDtypeStruct((_NW, _NH), jnp.float32),
    mesh=_mesh,
    compiler_params=_cp,
    scratch_types=[
        pltpu.VMEM((_NH,), jnp.float32),
        pltpu.VMEM((_NV, 16), jnp.int32),
    ],
)
def _deg_kernel(dst_hbm, out_hbm, hist, idx_v):
    c = lax.axis_index("c")
    s = lax.axis_index("s")
    wid = s * _NC + c
    pltpu.sync_copy(dst_hbm.at[wid], idx_v)

    @pl.loop(0, _NH, step=16)
    def _(i):
        hist[pl.ds(i, 16)] = jnp.zeros((16,), jnp.float32)

    @pl.loop(0, _NV)
    def _(j):
        v = idx_v[j, :]
        cnt, msk = plsc.scan_count(v)
        plsc.addupdate_scatter(hist, [v], cnt.astype(jnp.float32), mask=msk)

    pltpu.sync_copy(hist, out_hbm.at[wid])


@functools.partial(
    pl.kernel,
    out_type=jax.ShapeDtypeStruct((_NC, _NP, _D), jnp.float32),
    mesh=_mesh,
    scratch_types=[
        pltpu.VMEM_SHARED((_NP, _D), jnp.float32),
        pltpu.VMEM((8, _CH), jnp.int32),
        pltpu.VMEM((2, _CH, _D), jnp.float32),
        pltpu.SemaphoreType.DMA,
        pltpu.SemaphoreType.DMA((2,)),
        pltpu.SemaphoreType.DMA,
    ],
)
def _msg_kernel(h_hbm, src_hbm, dst_hbm, zeros_hbm, out_hbm,
                acc, idx_v, rows, isem, gsem, ssem):
    c = lax.axis_index("c")
    s = lax.axis_index("s")
    wid = s * _NC + c
    pltpu.sync_copy(zeros_hbm.at[pl.ds(s * _RPT, _RPT)],
                    acc.at[pl.ds(s * _RPT, _RPT)])
    plsc.subcore_barrier()

    @pl.loop(0, _NCH + 3)
    def _(jn):
        @pl.when(jnp.logical_and(jn >= 3, jn - 3 < _NCH))
        def _():
            j3 = jn - 3
            pltpu.make_async_copy(rows.at[lax.rem(j3, 2)],
                                  acc.at[idx_v.at[4 + lax.rem(j3, 4)]],
                                  ssem).wait()

        @pl.when(jn < _NCH)
        def _():
            sl = lax.rem(jn, 4)
            pltpu.async_copy(src_hbm.at[wid, jn], idx_v.at[sl], isem)
            pltpu.async_copy(dst_hbm.at[wid, jn], idx_v.at[4 + sl], isem)

        @pl.when(jnp.logical_and(jn >= 1, jn - 1 < _NCH))
        def _():
            j1 = jn - 1
            sl = lax.rem(j1, 4)
            pltpu.make_async_copy(src_hbm.at[wid, j1], idx_v.at[sl],
                                  isem).wait()
            pltpu.make_async_copy(dst_hbm.at[wid, j1], idx_v.at[4 + sl],
                                  isem).wait()
            pltpu.async_copy(h_hbm.at[idx_v.at[sl]], rows.at[lax.rem(j1, 2)],
                             gsem.at[lax.rem(j1, 2)])

        @pl.when(jnp.logical_and(jn >= 2, jn - 2 < _NCH))
        def _():
            j2 = jn - 2
            b = lax.rem(j2, 2)
            pltpu.make_async_copy(h_hbm.at[idx_v.at[lax.rem(j2, 4)]],
                                  rows.at[b], gsem.at[b]).wait()
            pltpu.async_copy(rows.at[b], acc.at[idx_v.at[4 + lax.rem(j2, 4)]],
                             ssem, add=True)

    plsc.subcore_barrier()
    pltpu.sync_copy(acc.at[pl.ds(s * _RPT, _RPT)],
                    out_hbm.at[c, pl.ds(s * _RPT, _RPT)])


_BN = 1280


def _dinv_block(degp):
    deg = jnp.sum(degp, axis=0)[:, None] + 1.0
    return lax.rsqrt(deg)


def _tc1_body(degp_ref, x_ref, w_ref, out_ref):
    dinv = _dinv_block(degp_ref[...])
    h = jnp.dot(x_ref[...], w_ref[...], preferred_element_type=jnp.float32)
    out_ref[...] = h * dinv


def _tc2_body(p_ref, h_ref, degp_ref, b_ref, w_ref, out_ref):
    dinv = _dinv_block(degp_ref[...])
    sarg = (p_ref[0] + p_ref[1] + h_ref[...]) * dinv + b_ref[...]
    r = jnp.maximum(sarg, 0.0)
    out_ref[...] = jnp.dot(r, w_ref[...],
                           preferred_element_type=jnp.float32) * dinv


def _tc3_body(p_ref, h_ref, degp_ref, b_ref, out_ref):
    dinv = _dinv_block(degp_ref[...])
    out_ref[...] = (p_ref[0] + p_ref[1] + h_ref[...]) * dinv + b_ref[...]


_degp_spec = pl.BlockSpec((_NW, _BN), lambda i: (0, i))
_row_spec = pl.BlockSpec((_BN, _D), lambda i: (i, 0))
_p_spec = pl.BlockSpec((_NC, _BN, _D), lambda i: (0, i, 0))
_w_spec = pl.BlockSpec((_D, _D), lambda i: (0, 0))
_b_spec = pl.BlockSpec((1, _D), lambda i: (0, 0))
_out_sds = jax.ShapeDtypeStruct((_N, _D), jnp.float32)

_tc1 = pl.pallas_call(
    _tc1_body,
    grid=(-(-_N // _BN),),
    in_specs=[_degp_spec, _row_spec, _w_spec],
    out_specs=_row_spec,
    out_shape=_out_sds,
)

_tc2 = pl.pallas_call(
    _tc2_body,
    grid=(-(-_N // _BN),),
    in_specs=[_p_spec, _row_spec, _degp_spec, _b_spec, _w_spec],
    out_specs=_row_spec,
    out_shape=_out_sds,
)

_tc3 = pl.pallas_call(
    _tc3_body,
    grid=(-(-_N // _BN),),
    in_specs=[_p_spec, _row_spec, _degp_spec, _b_spec],
    out_specs=_row_spec,
    out_shape=_out_sds,
)


def kernel(x, edge_index, W1, b1, W2, b2):
    pad = _EP - _E
    src = jnp.concatenate(
        [edge_index[0], jnp.zeros((pad,), jnp.int32)]).reshape(_NW, _NCH, _CH)
    dummy_dst = _N + (jnp.arange(pad, dtype=jnp.int32) % (_NP - _N))
    dst = jnp.concatenate(
        [edge_index[1], dummy_dst]).reshape(_NW, _NCH, _CH)
    zerosd = jnp.zeros((_NP, _D), jnp.float32)
    degp = _deg_kernel(edge_index[1].reshape(_NW, _NV, 16))
    h1 = _tc1(degp, x, W1)
    p1 = _msg_kernel(h1, src, dst, zerosd)
    h2 = _tc2(p1, h1, degp, b1.reshape(1, _D), W2)
    p2 = _msg_kernel(h2, src, dst, zerosd)
    return _tc3(p2, h2, degp, b2.reshape(1, _D))


# speedup vs baseline: 1.3630x; 1.3630x over previous
import dataclasses
import functools

import jax
import jax.numpy as jnp
from jax import lax
from jax.experimental import pallas as pl
from jax.experimental.pallas import tpu as pltpu
from jax.experimental.pallas import tpu_sc as plsc

_N = 10000
_D = 128
_E = 320000
_NC = 2
_NS = 16
_NW = _NC * _NS
_CH = 88
_NCH = 114
_EP = _NW * _NCH * _CH
_NP = 10112
_RPT = _NP // _NS

_mesh = plsc.VectorSubcoreMesh(core_axis_name="c", subcore_axis_name="s")
_NV = 625
_NH = 10240

_cp = pltpu.CompilerParams()
if "needs_layout_passes" in pltpu.CompilerParams.__dataclass_fields__:
    _cp = dataclasses.replace(_cp, needs_layout_passes=False)


@functools.partial(
    pl.kernel,
    out_type=jax.ShapeDtypeStruct((_NW, _NH), jnp.float32),
    mesh=_mesh,
    compiler_params=_cp,
    scratch_types=[
        pltpu.VMEM((_NH,), jnp.float32),
        pltpu.VMEM((_NV, 16), jnp.int32),
    ],
)
def _deg_kernel(dst_hbm, out_hbm, hist, idx_v):
    c = lax.axis_index("c")
    s = lax.axis_index("s")
    wid = s * _NC + c
    pltpu.sync_copy(dst_hbm.at[wid], idx_v)

    @pl.loop(0, _NH, step=16)
    def _(i):
        hist[pl.ds(i, 16)] = jnp.zeros((16,), jnp.float32)

    @pl.loop(0, _NV)
    def _(j):
        v = idx_v[j, :]
        cnt, msk = plsc.scan_count(v)
        plsc.addupdate_scatter(hist, [v], cnt.astype(jnp.float32), mask=msk)

    pltpu.sync_copy(hist, out_hbm.at[wid])


@functools.partial(
    pl.kernel,
    out_type=jax.ShapeDtypeStruct((_NC, _NP, _D), jnp.float32),
    mesh=_mesh,
    scratch_types=[
        pltpu.VMEM_SHARED((_NP, _D), jnp.float32),
        pltpu.VMEM((4, _CH), jnp.int32),
        pltpu.VMEM((4, _CH), jnp.int32),
        pltpu.VMEM((2, _CH, _D), jnp.float32),
        pltpu.SemaphoreType.DMA,
        pltpu.SemaphoreType.DMA((2,)),
        pltpu.SemaphoreType.DMA,
    ],
)
def _msg_kernel(h_hbm, src_hbm, dst_hbm, zeros_hbm, out_hbm,
                acc, src_v, dst_v, rows, isem, gsem, ssem):
    c = lax.axis_index("c")
    s = lax.axis_index("s")
    wid = s * _NC + c
    pltpu.sync_copy(zeros_hbm.at[pl.ds(s * _RPT, _RPT)],
                    acc.at[pl.ds(s * _RPT, _RPT)])
    plsc.subcore_barrier()

    @pl.loop(0, _NCH + 3)
    def _(jn):
        @pl.when(jnp.logical_and(jn >= 3, jn - 3 < _NCH))
        def _():
            j3 = jn - 3
            pltpu.make_async_copy(rows.at[lax.rem(j3, 2)],
                                  acc.at[dst_v.at[lax.rem(j3, 4)]],
                                  ssem).wait()

        @pl.when(jn < _NCH)
        def _():
            sl = lax.rem(jn, 4)
            pltpu.async_copy(src_hbm.at[wid, jn], src_v.at[sl], isem)
            pltpu.async_copy(dst_hbm.at[wid, jn], dst_v.at[sl], isem)

        @pl.when(jnp.logical_and(jn >= 1, jn - 1 < _NCH))
        def _():
            j1 = jn - 1
            sl = lax.rem(j1, 4)
            pltpu.make_async_copy(src_hbm.at[wid, j1], src_v.at[sl],
                                  isem).wait()
            pltpu.make_async_copy(dst_hbm.at[wid, j1], dst_v.at[sl],
                                  isem).wait()
            pltpu.async_copy(h_hbm.at[src_v.at[sl]], rows.at[lax.rem(j1, 2)],
                             gsem.at[lax.rem(j1, 2)])

        @pl.when(jnp.logical_and(jn >= 2, jn - 2 < _NCH))
        def _():
            j2 = jn - 2
            b = lax.rem(j2, 2)
            pltpu.make_async_copy(h_hbm.at[src_v.at[lax.rem(j2, 4)]],
                                  rows.at[b], gsem.at[b]).wait()
            pltpu.async_copy(rows.at[b], acc.at[dst_v.at[lax.rem(j2, 4)]],
                             ssem, add=True)

    plsc.subcore_barrier()
    pltpu.sync_copy(acc.at[pl.ds(s * _RPT, _RPT)],
                    out_hbm.at[c, pl.ds(s * _RPT, _RPT)])


_BN = 1280


def _dinv_block(degp):
    deg = jnp.sum(degp, axis=0)[:, None] + 1.0
    return lax.rsqrt(deg)


def _tc1_body(degp_ref, x_ref, w_ref, out_ref):
    dinv = _dinv_block(degp_ref[...])
    h = jnp.dot(x_ref[...], w_ref[...], preferred_element_type=jnp.float32)
    out_ref[...] = h * dinv


def _tc2_body(p_ref, h_ref, degp_ref, b_ref, w_ref, out_ref):
    dinv = _dinv_block(degp_ref[...])
    sarg = (p_ref[0] + p_ref[1] + h_ref[...]) * dinv + b_ref[...]
    r = jnp.maximum(sarg, 0.0)
    out_ref[...] = jnp.dot(r, w_ref[...],
                           preferred_element_type=jnp.float32) * dinv


def _tc3_body(p_ref, h_ref, degp_ref, b_ref, out_ref):
    dinv = _dinv_block(degp_ref[...])
    out_ref[...] = (p_ref[0] + p_ref[1] + h_ref[...]) * dinv + b_ref[...]


_degp_spec = pl.BlockSpec((_NW, _BN), lambda i: (0, i))
_row_spec = pl.BlockSpec((_BN, _D), lambda i: (i, 0))
_p_spec = pl.BlockSpec((_NC, _BN, _D), lambda i: (0, i, 0))
_w_spec = pl.BlockSpec((_D, _D), lambda i: (0, 0))
_b_spec = pl.BlockSpec((1, _D), lambda i: (0, 0))
_out_sds = jax.ShapeDtypeStruct((_N, _D), jnp.float32)

_tc1 = pl.pallas_call(
    _tc1_body,
    grid=(-(-_N // _BN),),
    in_specs=[_degp_spec, _row_spec, _w_spec],
    out_specs=_row_spec,
    out_shape=_out_sds,
)

_tc2 = pl.pallas_call(
    _tc2_body,
    grid=(-(-_N // _BN),),
    in_specs=[_p_spec, _row_spec, _degp_spec, _b_spec, _w_spec],
    out_specs=_row_spec,
    out_shape=_out_sds,
)

_tc3 = pl.pallas_call(
    _tc3_body,
    grid=(-(-_N // _BN),),
    in_specs=[_p_spec, _row_spec, _degp_spec, _b_spec],
    out_specs=_row_spec,
    out_shape=_out_sds,
)


def kernel(x, edge_index, W1, b1, W2, b2):
    pad = _EP - _E
    dummy_dst = _N + (jnp.arange(pad, dtype=jnp.int32) % (_NP - _N))
    src = jnp.concatenate(
        [edge_index[0], jnp.zeros((pad,), jnp.int32)]).reshape(_NW, _NCH, _CH)
    dst = jnp.concatenate(
        [edge_index[1], dummy_dst]).reshape(_NW, _NCH, _CH)
    zerosd = jnp.zeros((_NP, _D), jnp.float32)
    degp = _deg_kernel(edge_index[1].reshape(_NW, _NV, 16))
    h1 = _tc1(degp, x, W1)
    p1 = _msg_kernel(h1, src, dst, zerosd)
    h2 = _tc2(p1, h1, degp, b1.reshape(1, _D), W2)
    p2 = _msg_kernel(h2, src, dst, zerosd)
    return _tc3(p2, h2, degp, b2.reshape(1, _D))


# speedup vs baseline: 1.6883x; 1.2386x over previous
import dataclasses
import functools

import jax
import jax.numpy as jnp
from jax import lax
from jax.experimental import pallas as pl
from jax.experimental.pallas import tpu as pltpu
from jax.experimental.pallas import tpu_sc as plsc

_N = 10000
_D = 128
_E = 320000
_NC = 2
_NS = 16
_NW = _NC * _NS
_CH = 80
_NCH = 125
_NP = 10112
_RPT = _NP // _NS

_mesh = plsc.VectorSubcoreMesh(core_axis_name="c", subcore_axis_name="s")
_NV = 625
_NH = 10240

_cp = pltpu.CompilerParams()
if "needs_layout_passes" in pltpu.CompilerParams.__dataclass_fields__:
    _cp = dataclasses.replace(_cp, needs_layout_passes=False)


@functools.partial(
    pl.kernel,
    out_type=jax.ShapeDtypeStruct((_NW, _NH), jnp.float32),
    mesh=_mesh,
    compiler_params=_cp,
    scratch_types=[
        pltpu.VMEM((_NH,), jnp.float32),
        pltpu.VMEM((_NV, 16), jnp.int32),
    ],
)
def _deg_kernel(dst_hbm, out_hbm, hist, idx_v):
    c = lax.axis_index("c")
    s = lax.axis_index("s")
    wid = s * _NC + c
    pltpu.sync_copy(dst_hbm.at[wid], idx_v)

    @pl.loop(0, _NH, step=16)
    def _(i):
        hist[pl.ds(i, 16)] = jnp.zeros((16,), jnp.float32)

    @pl.loop(0, _NV)
    def _(j):
        v = idx_v[j, :]
        cnt, msk = plsc.scan_count(v)
        plsc.addupdate_scatter(hist, [v], cnt.astype(jnp.float32), mask=msk)

    pltpu.sync_copy(hist, out_hbm.at[wid])


@functools.partial(
    pl.kernel,
    out_type=jax.ShapeDtypeStruct((_NC, _NP, _D), jnp.float32),
    mesh=_mesh,
    scratch_types=[
        pltpu.VMEM_SHARED((_NP, _D), jnp.float32),
        pltpu.VMEM((4, _CH), jnp.int32),
        pltpu.VMEM((4, _CH), jnp.int32),
        pltpu.VMEM((2, _CH, _D), jnp.float32),
        pltpu.SemaphoreType.DMA,
        pltpu.SemaphoreType.DMA((2,)),
        pltpu.SemaphoreType.DMA,
    ],
)
def _msg_kernel(h_hbm, src_hbm, dst_hbm, zeros_hbm, out_hbm,
                acc, src_v, dst_v, rows, isem, gsem, ssem):
    c = lax.axis_index("c")
    s = lax.axis_index("s")
    wid = s * _NC + c
    pltpu.sync_copy(zeros_hbm.at[pl.ds(s * _RPT, _RPT)],
                    acc.at[pl.ds(s * _RPT, _RPT)])
    plsc.subcore_barrier()

    @pl.loop(0, _NCH + 3)
    def _(jn):
        @pl.when(jnp.logical_and(jn >= 3, jn - 3 < _NCH))
        def _():
            j3 = jn - 3
            pltpu.make_async_copy(rows.at[lax.rem(j3, 2)],
                                  acc.at[dst_v.at[lax.rem(j3, 4)]],
                                  ssem).wait()

        @pl.when(jn < _NCH)
        def _():
            sl = lax.rem(jn, 4)
            pltpu.async_copy(src_hbm.at[wid, jn], src_v.at[sl], isem)
            pltpu.async_copy(dst_hbm.at[wid, jn], dst_v.at[sl], isem)

        @pl.when(jnp.logical_and(jn >= 1, jn - 1 < _NCH))
        def _():
            j1 = jn - 1
            sl = lax.rem(j1, 4)
            pltpu.make_async_copy(src_hbm.at[wid, j1], src_v.at[sl],
                                  isem).wait()
            pltpu.make_async_copy(dst_hbm.at[wid, j1], dst_v.at[sl],
                                  isem).wait()
            pltpu.async_copy(h_hbm.at[src_v.at[sl]], rows.at[lax.rem(j1, 2)],
                             gsem.at[lax.rem(j1, 2)])

        @pl.when(jnp.logical_and(jn >= 2, jn - 2 < _NCH))
        def _():
            j2 = jn - 2
            b = lax.rem(j2, 2)
            pltpu.make_async_copy(h_hbm.at[src_v.at[lax.rem(j2, 4)]],
                                  rows.at[b], gsem.at[b]).wait()
            pltpu.async_copy(rows.at[b], acc.at[dst_v.at[lax.rem(j2, 4)]],
                             ssem, add=True)

    plsc.subcore_barrier()
    pltpu.sync_copy(acc.at[pl.ds(s * _RPT, _RPT)],
                    out_hbm.at[c, pl.ds(s * _RPT, _RPT)])


_BN = 1280


def _dinv_block(degp):
    deg = jnp.sum(degp, axis=0)[:, None] + 1.0
    return lax.rsqrt(deg)


def _tc1_body(degp_ref, x_ref, w_ref, out_ref):
    dinv = _dinv_block(degp_ref[...])
    h = jnp.dot(x_ref[...], w_ref[...], preferred_element_type=jnp.float32)
    out_ref[...] = h * dinv


def _tc2_body(p_ref, h_ref, degp_ref, b_ref, w_ref, out_ref):
    dinv = _dinv_block(degp_ref[...])
    sarg = (p_ref[0] + p_ref[1] + h_ref[...]) * dinv + b_ref[...]
    r = jnp.maximum(sarg, 0.0)
    out_ref[...] = jnp.dot(r, w_ref[...],
                           preferred_element_type=jnp.float32) * dinv


def _tc3_body(p_ref, h_ref, degp_ref, b_ref, out_ref):
    dinv = _dinv_block(degp_ref[...])
    out_ref[...] = (p_ref[0] + p_ref[1] + h_ref[...]) * dinv + b_ref[...]


_degp_spec = pl.BlockSpec((_NW, _BN), lambda i: (0, i))
_row_spec = pl.BlockSpec((_BN, _D), lambda i: (i, 0))
_p_spec = pl.BlockSpec((_NC, _BN, _D), lambda i: (0, i, 0))
_w_spec = pl.BlockSpec((_D, _D), lambda i: (0, 0))
_b_spec = pl.BlockSpec((1, _D), lambda i: (0, 0))
_out_sds = jax.ShapeDtypeStruct((_N, _D), jnp.float32)

_tc1 = pl.pallas_call(
    _tc1_body,
    grid=(-(-_N // _BN),),
    in_specs=[_degp_spec, _row_spec, _w_spec],
    out_specs=_row_spec,
    out_shape=_out_sds,
)

_tc2 = pl.pallas_call(
    _tc2_body,
    grid=(-(-_N // _BN),),
    in_specs=[_p_spec, _row_spec, _degp_spec, _b_spec, _w_spec],
    out_specs=_row_spec,
    out_shape=_out_sds,
)

_tc3 = pl.pallas_call(
    _tc3_body,
    grid=(-(-_N // _BN),),
    in_specs=[_p_spec, _row_spec, _degp_spec, _b_spec],
    out_specs=_row_spec,
    out_shape=_out_sds,
)


def kernel(x, edge_index, W1, b1, W2, b2):
    src = edge_index[0].reshape(_NW, _NCH, _CH)
    dst = edge_index[1].reshape(_NW, _NCH, _CH)
    zerosd = jnp.zeros((_NP, _D), jnp.float32)
    degp = _deg_kernel(edge_index[1].reshape(_NW, _NV, 16))
    h1 = _tc1(degp, x, W1)
    p1 = _msg_kernel(h1, src, dst, zerosd)
    h2 = _tc2(p1, h1, degp, b1.reshape(1, _D), W2)
    p2 = _msg_kernel(h2, src, dst, zerosd)
    return _tc3(p2, h2, degp, b2.reshape(1, _D))
